# Initial kernel scaffold; baseline (speedup 1.0000x reference)
import jax
import jax.numpy as jnp
from jax.experimental import pallas as pl


def kernel(token_types, card_uid_indices, status_uid_indices, enemy_intent_indices, encoded_numbers, tok_table, card_table, status_table, intent_table, W1, b1, W2, b2):
    raise NotImplementedError("write your pallas kernel here")


# speedup vs baseline: 5.0803x; 5.0803x over previous
import functools

import jax
import jax.numpy as jnp
from jax import lax
from jax.experimental import pallas as pl
from jax.experimental.pallas import tpu as pltpu
from jax.experimental.pallas import tpu_sc as plsc

MAX_ENCODED_NUMBER = 999.0
MAX_SEQ_LEN = 128
NUM_ROWS = 1024

_NC, _NS = 2, 16
_NW = _NC * _NS


def _mask_and_table_body(tok_ref, w1_ref, b1_ref, w2_ref, b2_ref,
                         mask_ref, num_ref):
    i = pl.program_id(0)
    mask_ref[...] = tok_ref[...] == 0

    @pl.when(i == 0)
    def _():
        x = lax.broadcasted_iota(jnp.int32, (NUM_ROWS, 1), 0).astype(
            jnp.float32) * (1.0 / MAX_ENCODED_NUMBER)
        h = jnp.maximum(x * w1_ref[...] + b1_ref[...], 0.0)
        y = jnp.dot(h, w2_ref[...], preferred_element_type=jnp.float32)
        num_ref[...] = jnp.maximum(y + b2_ref[...], 0.0)


def _mask_and_num_table(token_types, W1, b1, W2, b2):
    B, S = token_types.shape
    blk = 512 if B % 512 == 0 else B
    grid = (B // blk,)
    return pl.pallas_call(
        _mask_and_table_body,
        grid=grid,
        in_specs=[
            pl.BlockSpec((blk, S), lambda i: (i, 0)),
            pl.BlockSpec((1, 32), lambda i: (0, 0)),
            pl.BlockSpec((1, 32), lambda i: (0, 0)),
            pl.BlockSpec((32, 64), lambda i: (0, 0)),
            pl.BlockSpec((1, 64), lambda i: (0, 0)),
        ],
        out_specs=[
            pl.BlockSpec((blk, S), lambda i: (i, 0)),
            pl.BlockSpec((NUM_ROWS, 64), lambda i: (0, 0)),
        ],
        out_shape=[
            jax.ShapeDtypeStruct((B, S), jnp.bool_),
            jax.ShapeDtypeStruct((NUM_ROWS, 64), jnp.float32),
        ],
    )(token_types, W1.reshape(1, 32), b1.reshape(1, 32), W2,
      b2.reshape(1, 64))


def _round_up(n, m):
    return (n + m - 1) // m * m


def _pack_tables(tables):
    parts, bases, off = [], [], 0
    for t in tables:
        f = t.reshape(-1)
        n = _round_up(f.shape[0], 128)
        parts.append(jnp.pad(f, (0, n - f.shape[0])))
        bases.append(off)
        off += n
    return jnp.concatenate(parts), bases, off


def _sc_embed(N, CHUNK, packed_len, bases):
    rows_per_w = N // _NW
    n_chunks = rows_per_w // CHUNK
    b_tok, b_card, b_status, b_intent, b_num = bases
    mesh = plsc.VectorSubcoreMesh(core_axis_name="c", subcore_axis_name="s")

    @functools.partial(
        pl.kernel,
        out_type=jax.ShapeDtypeStruct((N, 160), jnp.float32),
        mesh=mesh,
        scratch_types=[
            pltpu.VMEM((packed_len,), jnp.float32),
            pltpu.VMEM((CHUNK,), jnp.int32),
            pltpu.VMEM((CHUNK,), jnp.int32),
            pltpu.VMEM((CHUNK,), jnp.int32),
            pltpu.VMEM((CHUNK,), jnp.int32),
            pltpu.VMEM((CHUNK,), jnp.int32),
            pltpu.VMEM((CHUNK, 160), jnp.float32),
            pltpu.SemaphoreType.DMA,
        ],
    )
    def body(tok_i, card_i, status_i, intent_i, num_i, packed, out,
             T, tok_iv, card_iv, status_iv, intent_iv, num_iv, stage, sem):
        wid = lax.axis_index("s") * _NC + lax.axis_index("c")
        pltpu.sync_copy(packed, T)

        def chunk_step(ci, carry):
            base = wid * rows_per_w + ci * CHUNK
            sl = pl.ds(base, CHUNK)
            pltpu.sync_copy(tok_i.at[sl], tok_iv)
            pltpu.sync_copy(card_i.at[sl], card_iv)
            pltpu.sync_copy(status_i.at[sl], status_iv)
            pltpu.sync_copy(intent_i.at[sl], intent_iv)
            pltpu.sync_copy(num_i.at[sl], num_iv)

            def group_step(g, c2):
                r0 = g * 16
                tok_v = tok_iv[pl.ds(r0, 16)] * 32 + b_tok
                card_v = card_iv[pl.ds(r0, 16)] * 32 + b_card
                status_v = status_iv[pl.ds(r0, 16)] * 16 + b_status
                intent_v = intent_iv[pl.ds(r0, 16)] * 16 + b_intent
                num_v = num_iv[pl.ds(r0, 16)] * 64 + b_num
                for j in range(16):
                    r = r0 + j
                    o_tok = tok_v[j]
                    o_card = card_v[j]
                    o_num = num_v[j]
                    stage[r, pl.ds(0, 16)] = T[pl.ds(o_tok, 16)]
                    stage[r, pl.ds(16, 16)] = T[pl.ds(o_tok + 16, 16)]
                    stage[r, pl.ds(32, 16)] = T[pl.ds(o_card, 16)]
                    stage[r, pl.ds(48, 16)] = T[pl.ds(o_card + 16, 16)]
                    stage[r, pl.ds(64, 16)] = T[pl.ds(status_v[j], 16)]
                    stage[r, pl.ds(80, 16)] = T[pl.ds(intent_v[j], 16)]
                    stage[r, pl.ds(96, 16)] = T[pl.ds(o_num, 16)]
                    stage[r, pl.ds(112, 16)] = T[pl.ds(o_num + 16, 16)]
                    stage[r, pl.ds(128, 16)] = T[pl.ds(o_num + 32, 16)]
                    stage[r, pl.ds(144, 16)] = T[pl.ds(o_num + 48, 16)]
                return c2

            lax.fori_loop(0, CHUNK // 16, group_step, 0)
            pltpu.sync_copy(stage, out.at[sl, :])
            return carry

        lax.fori_loop(0, n_chunks, chunk_step, 0)

    return body


def kernel(token_types, card_uid_indices, status_uid_indices,
           enemy_intent_indices, encoded_numbers, tok_table, card_table,
           status_table, intent_table, W1, b1, W2, b2):
    seq_len = min(token_types.shape[-1], MAX_SEQ_LEN)
    if token_types.ndim == 1:
        token_types = token_types[None, :]
        card_uid_indices = card_uid_indices[None, :]
        status_uid_indices = status_uid_indices[None, :]
        enemy_intent_indices = enemy_intent_indices[None, :]
        encoded_numbers = encoded_numbers[None, :]
    token_types = token_types[:, :seq_len]
    card_uid_indices = card_uid_indices[:, :seq_len]
    status_uid_indices = status_uid_indices[:, :seq_len]
    enemy_intent_indices = enemy_intent_indices[:, :seq_len]
    encoded_numbers = encoded_numbers[:, :seq_len]

    B, S = token_types.shape
    N = B * S
    mask, num_table = _mask_and_num_table(token_types, W1, b1, W2, b2)
    packed, bases, packed_len = _pack_tables(
        [tok_table, card_table, status_table, intent_table, num_table])

    i32 = jnp.int32
    out_flat = _sc_embed(N, 128, packed_len, bases)(
        token_types.reshape(N).astype(i32),
        card_uid_indices.reshape(N).astype(i32),
        status_uid_indices.reshape(N).astype(i32),
        enemy_intent_indices.reshape(N).astype(i32),
        encoded_numbers.reshape(N).astype(i32),
        packed)
    return out_flat.reshape(B, S, 160), mask


# speedup vs baseline: 7.5061x; 1.4775x over previous
import functools

import jax
import jax.numpy as jnp
from jax import lax
from jax.experimental import pallas as pl
from jax.experimental.pallas import tpu as pltpu
from jax.experimental.pallas import tpu_sc as plsc

MAX_ENCODED_NUMBER = 999.0
MAX_SEQ_LEN = 128
NUM_ROWS = 1024

_NC, _NS = 2, 16
_NW = _NC * _NS


def _mask_and_table_body(tok_ref, w1_ref, b1_ref, w2_ref, b2_ref,
                         mask_ref, num_ref):
    i = pl.program_id(0)
    mask_ref[...] = tok_ref[...] == 0

    @pl.when(i == 0)
    def _():
        x = lax.broadcasted_iota(jnp.int32, (NUM_ROWS, 1), 0).astype(
            jnp.float32) * (1.0 / MAX_ENCODED_NUMBER)
        h = jnp.maximum(x * w1_ref[...] + b1_ref[...], 0.0)
        y = jnp.dot(h, w2_ref[...], preferred_element_type=jnp.float32)
        num_ref[...] = jnp.maximum(y + b2_ref[...], 0.0)


def _mask_and_num_table(token_types, W1, b1, W2, b2):
    B, S = token_types.shape
    blk = 512 if B % 512 == 0 else B
    grid = (B // blk,)
    return pl.pallas_call(
        _mask_and_table_body,
        grid=grid,
        in_specs=[
            pl.BlockSpec((blk, S), lambda i: (i, 0)),
            pl.BlockSpec((1, 32), lambda i: (0, 0)),
            pl.BlockSpec((1, 32), lambda i: (0, 0)),
            pl.BlockSpec((32, 64), lambda i: (0, 0)),
            pl.BlockSpec((1, 64), lambda i: (0, 0)),
        ],
        out_specs=[
            pl.BlockSpec((blk, S), lambda i: (i, 0)),
            pl.BlockSpec((NUM_ROWS, 64), lambda i: (0, 0)),
        ],
        out_shape=[
            jax.ShapeDtypeStruct((B, S), jnp.bool_),
            jax.ShapeDtypeStruct((NUM_ROWS, 64), jnp.float32),
        ],
    )(token_types, W1.reshape(1, 32), b1.reshape(1, 32), W2,
      b2.reshape(1, 64))


def _round_up(n, m):
    return (n + m - 1) // m * m


def _pack_tables(tables):
    parts, bases, off = [], [], 0
    for t in tables:
        f = t.reshape(-1)
        n = _round_up(f.shape[0], 128)
        parts.append(jnp.pad(f, (0, n - f.shape[0])))
        bases.append(off)
        off += n
    return jnp.concatenate(parts), bases, off


def _sc_embed(N, CHUNK, packed_len, bases):
    rows_per_w = N // _NW
    n_chunks = rows_per_w // CHUNK
    b_tok, b_card, b_status, b_intent, b_num = bases
    mesh = plsc.VectorSubcoreMesh(core_axis_name="c", subcore_axis_name="s")

    @functools.partial(
        pl.kernel,
        out_type=jax.ShapeDtypeStruct((N, 160), jnp.float32),
        mesh=mesh,
        scratch_types=[
            pltpu.VMEM((packed_len,), jnp.float32),
            pltpu.VMEM((2, 5, CHUNK), jnp.int32),
            pltpu.VMEM((2, CHUNK, 160), jnp.float32),
            pltpu.SemaphoreType.DMA,
            pltpu.SemaphoreType.DMA,
            pltpu.SemaphoreType.DMA,
        ],
    )
    def body(tok_i, card_i, status_i, intent_i, num_i, packed, out,
             T, idx_v, stage, isem, wsem0, wsem1):
        wid = lax.axis_index("s") * _NC + lax.axis_index("c")
        w_base = wid * rows_per_w
        idx_refs = (tok_i, card_i, status_i, intent_i, num_i)
        pltpu.sync_copy(packed, T)

        def issue_idx(ci, pb):
            sl = pl.ds(w_base + ci * CHUNK, CHUNK)
            for f, ref in enumerate(idx_refs):
                pltpu.async_copy(ref.at[sl], idx_v.at[pb, f], isem)

        def wait_idx(pb):
            sl = pl.ds(w_base, CHUNK)
            for f, ref in enumerate(idx_refs):
                pltpu.make_async_copy(ref.at[sl], idx_v.at[pb, f], isem).wait()

        def wait_write(pb, sem):
            pltpu.make_async_copy(out.at[pl.ds(0, CHUNK), :],
                                  stage.at[pb], sem).wait()

        issue_idx(0, 0)

        def chunk_step(ci, carry):
            cb = lax.rem(ci, 2)
            base = w_base + ci * CHUNK
            sl = pl.ds(base, CHUNK)
            wait_idx(cb)

            @pl.when(ci + 1 < n_chunks)
            def _():
                issue_idx(ci + 1, 1 - cb)

            @pl.when((ci >= 2) & (cb == 0))
            def _():
                wait_write(cb, wsem0)

            @pl.when((ci >= 2) & (cb == 1))
            def _():
                wait_write(cb, wsem1)

            def group_step(g, c2):
                r0 = g * 16
                tok_v = idx_v[cb, 0, pl.ds(r0, 16)] * 32 + b_tok
                card_v = idx_v[cb, 1, pl.ds(r0, 16)] * 32 + b_card
                status_v = idx_v[cb, 2, pl.ds(r0, 16)] * 16 + b_status
                intent_v = idx_v[cb, 3, pl.ds(r0, 16)] * 16 + b_intent
                num_v = idx_v[cb, 4, pl.ds(r0, 16)] * 64 + b_num
                for j in range(16):
                    r = r0 + j
                    o_tok = tok_v[j]
                    o_card = card_v[j]
                    o_num = num_v[j]
                    stage[cb, r, pl.ds(0, 16)] = T[pl.ds(o_tok, 16)]
                    stage[cb, r, pl.ds(16, 16)] = T[pl.ds(o_tok + 16, 16)]
                    stage[cb, r, pl.ds(32, 16)] = T[pl.ds(o_card, 16)]
                    stage[cb, r, pl.ds(48, 16)] = T[pl.ds(o_card + 16, 16)]
                    stage[cb, r, pl.ds(64, 16)] = T[pl.ds(status_v[j], 16)]
                    stage[cb, r, pl.ds(80, 16)] = T[pl.ds(intent_v[j], 16)]
                    stage[cb, r, pl.ds(96, 16)] = T[pl.ds(o_num, 16)]
                    stage[cb, r, pl.ds(112, 16)] = T[pl.ds(o_num + 16, 16)]
                    stage[cb, r, pl.ds(128, 16)] = T[pl.ds(o_num + 32, 16)]
                    stage[cb, r, pl.ds(144, 16)] = T[pl.ds(o_num + 48, 16)]
                return c2

            lax.fori_loop(0, CHUNK // 16, group_step, 0)

            @pl.when(cb == 0)
            def _():
                pltpu.async_copy(stage.at[0], out.at[sl, :], wsem0)

            @pl.when(cb == 1)
            def _():
                pltpu.async_copy(stage.at[1], out.at[sl, :], wsem1)

            return carry

        lax.fori_loop(0, n_chunks, chunk_step, 0)
        wait_write(0, wsem0)
        wait_write(1, wsem1)

    return body


def kernel(token_types, card_uid_indices, status_uid_indices,
           enemy_intent_indices, encoded_numbers, tok_table, card_table,
           status_table, intent_table, W1, b1, W2, b2):
    seq_len = min(token_types.shape[-1], MAX_SEQ_LEN)
    if token_types.ndim == 1:
        token_types = token_types[None, :]
        card_uid_indices = card_uid_indices[None, :]
        status_uid_indices = status_uid_indices[None, :]
        enemy_intent_indices = enemy_intent_indices[None, :]
        encoded_numbers = encoded_numbers[None, :]
    token_types = token_types[:, :seq_len]
    card_uid_indices = card_uid_indices[:, :seq_len]
    status_uid_indices = status_uid_indices[:, :seq_len]
    enemy_intent_indices = enemy_intent_indices[:, :seq_len]
    encoded_numbers = encoded_numbers[:, :seq_len]

    B, S = token_types.shape
    N = B * S
    mask, num_table = _mask_and_num_table(token_types, W1, b1, W2, b2)
    packed, bases, packed_len = _pack_tables(
        [tok_table, card_table, status_table, intent_table, num_table])

    i32 = jnp.int32
    out_flat = _sc_embed(N, 64, packed_len, bases)(
        token_types.reshape(N).astype(i32),
        card_uid_indices.reshape(N).astype(i32),
        status_uid_indices.reshape(N).astype(i32),
        enemy_intent_indices.reshape(N).astype(i32),
        encoded_numbers.reshape(N).astype(i32),
        packed)
    return out_flat.reshape(B, S, 160), mask


# speedup vs baseline: 12.3929x; 1.6510x over previous
import functools

import jax
import jax.numpy as jnp
from jax import lax
from jax.experimental import pallas as pl
from jax.experimental.pallas import tpu as pltpu
from jax.experimental.pallas import tpu_sc as plsc

MAX_ENCODED_NUMBER = 999.0
MAX_SEQ_LEN = 128
NUM_ROWS = 1024

_NC, _NS = 2, 16
_NW = _NC * _NS


def _mask_and_table_body(tok_ref, w1_ref, b1_ref, w2_ref, b2_ref,
                         mask_ref, num_ref):
    i = pl.program_id(0)
    mask_ref[...] = tok_ref[...] == 0

    @pl.when(i == 0)
    def _():
        x = lax.broadcasted_iota(jnp.int32, (NUM_ROWS, 1), 0).astype(
            jnp.float32) * (1.0 / MAX_ENCODED_NUMBER)
        h = jnp.maximum(x * w1_ref[...] + b1_ref[...], 0.0)
        y = jnp.dot(h, w2_ref[...], preferred_element_type=jnp.float32)
        num_ref[...] = jnp.maximum(y + b2_ref[...], 0.0)


def _mask_and_num_table(token_types, W1, b1, W2, b2):
    B, S = token_types.shape
    blk = 512 if B % 512 == 0 else B
    grid = (B // blk,)
    return pl.pallas_call(
        _mask_and_table_body,
        grid=grid,
        in_specs=[
            pl.BlockSpec((blk, S), lambda i: (i, 0)),
            pl.BlockSpec((1, 32), lambda i: (0, 0)),
            pl.BlockSpec((1, 32), lambda i: (0, 0)),
            pl.BlockSpec((32, 64), lambda i: (0, 0)),
            pl.BlockSpec((1, 64), lambda i: (0, 0)),
        ],
        out_specs=[
            pl.BlockSpec((blk, S), lambda i: (i, 0)),
            pl.BlockSpec((NUM_ROWS, 64), lambda i: (0, 0)),
        ],
        out_shape=[
            jax.ShapeDtypeStruct((B, S), jnp.bool_),
            jax.ShapeDtypeStruct((NUM_ROWS, 64), jnp.float32),
        ],
    )(token_types, W1.reshape(1, 32), b1.reshape(1, 32), W2,
      b2.reshape(1, 64))


def _round_up(n, m):
    return (n + m - 1) // m * m


def _pack_tables(tables):
    parts, bases, off = [], [], 0
    for t in tables:
        f = t.reshape(-1)
        n = _round_up(f.shape[0], 128)
        parts.append(jnp.pad(f, (0, n - f.shape[0])))
        bases.append(off)
        off += n
    return jnp.concatenate(parts), bases, off


def _sc_embed(N, CHUNK, packed_len, bases):
    rows_per_w = N // _NW
    n_chunks = rows_per_w // CHUNK
    b_tok, b_card, b_status, b_intent, b_num = bases
    mesh = plsc.VectorSubcoreMesh(core_axis_name="c", subcore_axis_name="s")

    @functools.partial(
        pl.kernel,
        out_type=jax.ShapeDtypeStruct((N, 160), jnp.float32),
        mesh=mesh,
        scratch_types=[
            pltpu.VMEM((packed_len,), jnp.float32),
            pltpu.VMEM((2, 5, CHUNK), jnp.int32),
            pltpu.VMEM((2, CHUNK, 160), jnp.float32),
            pltpu.SemaphoreType.DMA,
            pltpu.SemaphoreType.DMA,
            pltpu.SemaphoreType.DMA,
        ],
    )
    def body(tok_i, card_i, status_i, intent_i, num_i, packed, out,
             T, idx_v, stage, isem, wsem0, wsem1):
        wid = lax.axis_index("s") * _NC + lax.axis_index("c")
        w_base = wid * rows_per_w
        idx_refs = (tok_i, card_i, status_i, intent_i, num_i)
        pltpu.sync_copy(packed, T)

        def issue_idx(ci, pb):
            sl = pl.ds(w_base + ci * CHUNK, CHUNK)
            for f, ref in enumerate(idx_refs):
                pltpu.async_copy(ref.at[sl], idx_v.at[pb, f], isem)

        def wait_idx(pb):
            sl = pl.ds(w_base, CHUNK)
            for f, ref in enumerate(idx_refs):
                pltpu.make_async_copy(ref.at[sl], idx_v.at[pb, f], isem).wait()

        def wait_write(pb, sem):
            pltpu.make_async_copy(out.at[pl.ds(0, CHUNK), :],
                                  stage.at[pb], sem).wait()

        issue_idx(0, 0)

        def chunk_step(ci, carry):
            cb = lax.rem(ci, 2)
            base = w_base + ci * CHUNK
            sl = pl.ds(base, CHUNK)
            wait_idx(cb)

            @pl.when(ci + 1 < n_chunks)
            def _():
                issue_idx(ci + 1, 1 - cb)

            @pl.when((ci >= 2) & (cb == 0))
            def _():
                wait_write(cb, wsem0)

            @pl.when((ci >= 2) & (cb == 1))
            def _():
                wait_write(cb, wsem1)

            def group_step(g, c2):
                r0 = g * 16
                tok_v = idx_v[cb, 0, pl.ds(r0, 16)] * 32 + b_tok
                card_v = idx_v[cb, 1, pl.ds(r0, 16)] * 32 + b_card
                status_v = idx_v[cb, 2, pl.ds(r0, 16)] * 16 + b_status
                intent_v = idx_v[cb, 3, pl.ds(r0, 16)] * 16 + b_intent
                num_v = idx_v[cb, 4, pl.ds(r0, 16)] * 64 + b_num
                def row_loads(j):
                    o_tok = tok_v[j]
                    o_card = card_v[j]
                    o_num = num_v[j]
                    return (T[pl.ds(o_tok, 16)], T[pl.ds(o_tok + 16, 16)],
                            T[pl.ds(o_card, 16)], T[pl.ds(o_card + 16, 16)],
                            T[pl.ds(status_v[j], 16)],
                            T[pl.ds(intent_v[j], 16)],
                            T[pl.ds(o_num, 16)], T[pl.ds(o_num + 16, 16)],
                            T[pl.ds(o_num + 32, 16)],
                            T[pl.ds(o_num + 48, 16)])

                def row_stores(j, vals):
                    r = r0 + j
                    for p, v in enumerate(vals):
                        stage[cb, r, pl.ds(p * 16, 16)] = v

                prev = row_loads(0)
                for j in range(1, 16):
                    cur = row_loads(j)
                    row_stores(j - 1, prev)
                    prev = cur
                row_stores(15, prev)
                return c2

            lax.fori_loop(0, CHUNK // 16, group_step, 0)

            @pl.when(cb == 0)
            def _():
                pltpu.async_copy(stage.at[0], out.at[sl, :], wsem0)

            @pl.when(cb == 1)
            def _():
                pltpu.async_copy(stage.at[1], out.at[sl, :], wsem1)

            return carry

        lax.fori_loop(0, n_chunks, chunk_step, 0)
        wait_write(0, wsem0)
        wait_write(1, wsem1)

    return body


def kernel(token_types, card_uid_indices, status_uid_indices,
           enemy_intent_indices, encoded_numbers, tok_table, card_table,
           status_table, intent_table, W1, b1, W2, b2):
    seq_len = min(token_types.shape[-1], MAX_SEQ_LEN)
    if token_types.ndim == 1:
        token_types = token_types[None, :]
        card_uid_indices = card_uid_indices[None, :]
        status_uid_indices = status_uid_indices[None, :]
        enemy_intent_indices = enemy_intent_indices[None, :]
        encoded_numbers = encoded_numbers[None, :]
    token_types = token_types[:, :seq_len]
    card_uid_indices = card_uid_indices[:, :seq_len]
    status_uid_indices = status_uid_indices[:, :seq_len]
    enemy_intent_indices = enemy_intent_indices[:, :seq_len]
    encoded_numbers = encoded_numbers[:, :seq_len]

    B, S = token_types.shape
    N = B * S
    mask, num_table = _mask_and_num_table(token_types, W1, b1, W2, b2)
    packed, bases, packed_len = _pack_tables(
        [tok_table, card_table, status_table, intent_table, num_table])

    i32 = jnp.int32
    out_flat = _sc_embed(N, 64, packed_len, bases)(
        token_types.reshape(N).astype(i32),
        card_uid_indices.reshape(N).astype(i32),
        status_uid_indices.reshape(N).astype(i32),
        enemy_intent_indices.reshape(N).astype(i32),
        encoded_numbers.reshape(N).astype(i32),
        packed)
    return out_flat.reshape(B, S, 160), mask
